# Initial kernel scaffold; baseline (speedup 1.0000x reference)
import functools

import jax
import jax.numpy as jnp
from jax import lax
from jax.experimental import pallas as pl
from jax.experimental.pallas import tpu as pltpu
from jax.experimental.pallas import tpu_sc as plsc

N = 320000
D = 128
B = 1024

BA = 8192
BN = 2048
NSC = 16
CHUNK = N // NSC
ITERS = CHUNK // 16


def _gate_kernel(wg_ref, bg_ref, h_ref, gate_ref, gmax_ref):
    g = jnp.sum(h_ref[...] * wg_ref[...], axis=1, keepdims=True) + bg_ref[0, 0]
    gate_ref[...] = g
    bm = jnp.max(g)

    @pl.when(pl.program_id(0) == 0)
    def _():
        gmax_ref[0, 0] = bm

    @pl.when(pl.program_id(0) > 0)
    def _():
        gmax_ref[0, 0] = jnp.maximum(gmax_ref[0, 0], bm)


def _gate_pass(h, wg_row, bg2):
    return pl.pallas_call(
        _gate_kernel,
        grid=(N // BA,),
        in_specs=[
            pl.BlockSpec((1, D), lambda i: (0, 0)),
            pl.BlockSpec(memory_space=pltpu.SMEM),
            pl.BlockSpec((BA, D), lambda i: (i, 0)),
        ],
        out_specs=[
            pl.BlockSpec((BA, 1), lambda i: (i, 0)),
            pl.BlockSpec(memory_space=pltpu.SMEM),
        ],
        out_shape=[
            jax.ShapeDtypeStruct((N, 1), jnp.float32),
            jax.ShapeDtypeStruct((1, 1), jnp.float32),
        ],
    )(wg_row, bg2, h)


def _alpha_kernel(gate_hbm, batch_hbm, gmax_hbm, alpha_hbm,
                  g_v, b_v, sloc_v, stot_v, sall_v, gm_v, sh_shared):
    sid = lax.axis_index("s")
    base = sid * CHUNK
    pltpu.sync_copy(gate_hbm.at[pl.ds(base, CHUNK)], g_v)
    pltpu.sync_copy(batch_hbm.at[pl.ds(base, CHUNK)], b_v)
    pltpu.sync_copy(gmax_hbm, gm_v)
    gmax = gm_v[...]

    zeros = jnp.zeros((16,), jnp.float32)

    def zbody(j, carry):
        sloc_v[pl.ds(j * 16, 16)] = zeros
        return carry

    lax.fori_loop(0, B // 16, zbody, 0)

    def body1(i, carry):
        off = i * 16
        e = jnp.exp(g_v[pl.ds(off, 16)] - gmax)
        ids = b_v[pl.ds(off, 16)]
        g_v[pl.ds(off, 16)] = e
        plsc.addupdate_scatter(sloc_v, [ids], e)
        return carry

    lax.fori_loop(0, ITERS, body1, 0)

    pltpu.sync_copy(sloc_v, sh_shared.at[sid])
    plsc.subcore_barrier()
    pltpu.sync_copy(sh_shared, sall_v)

    def cbody(j, carry):
        off = j * 16
        acc = sall_v[0, pl.ds(off, 16)]
        for w in range(1, NSC):
            acc = acc + sall_v[w, pl.ds(off, 16)]
        stot_v[pl.ds(off, 16)] = acc
        return carry

    lax.fori_loop(0, B // 16, cbody, 0)

    def body2(i, carry):
        off = i * 16
        e = g_v[pl.ds(off, 16)]
        ids = b_v[pl.ds(off, 16)]
        s = plsc.load_gather(stot_v, [ids])
        g_v[pl.ds(off, 16)] = e / (s + 1e-8)
        return carry

    lax.fori_loop(0, ITERS, body2, 0)

    pltpu.sync_copy(g_v, alpha_hbm.at[pl.ds(base, CHUNK)])


_alpha_pass = functools.partial(
    pl.kernel,
    mesh=plsc.VectorSubcoreMesh(core_axis_name="c", subcore_axis_name="s",
                                num_cores=1),
    out_type=jax.ShapeDtypeStruct((N,), jnp.float32),
    scratch_types=[
        pltpu.VMEM((CHUNK,), jnp.float32),
        pltpu.VMEM((CHUNK,), jnp.int32),
        pltpu.VMEM((B,), jnp.float32),
        pltpu.VMEM((B,), jnp.float32),
        pltpu.VMEM((NSC, B), jnp.float32),
        pltpu.VMEM((16,), jnp.float32),
        pltpu.VMEM_SHARED((NSC, B), jnp.float32),
    ],
)(_alpha_kernel)


def _pool_kernel(ids_ref, alpha_ref, h_ref, out_ref):
    @pl.when(pl.program_id(0) == 0)
    def _():
        out_ref[...] = jnp.zeros_like(out_ref)

    w = h_ref[...] * alpha_ref[...]
    ids = ids_ref[...]
    lo = jnp.min(ids) // 128
    hi = jnp.max(ids) // 128
    iota = lax.broadcasted_iota(jnp.int32, (BN, 128), 1)

    def chunk_body(k, carry):
        seg0 = (lo + k) * 128
        m = (ids == (iota + seg0)).astype(jnp.float32)
        part = lax.dot_general(m, w, (((0,), (0,)), ((), ())),
                               preferred_element_type=jnp.float32)
        out_ref[pl.ds(seg0, 128), :] += part
        return carry

    lax.fori_loop(0, hi - lo + 1, chunk_body, 0)


def _pool_pass(ids2, alpha2, h):
    return pl.pallas_call(
        _pool_kernel,
        grid=(N // BN,),
        in_specs=[
            pl.BlockSpec((BN, 1), lambda i: (i, 0)),
            pl.BlockSpec((BN, 1), lambda i: (i, 0)),
            pl.BlockSpec((BN, D), lambda i: (i, 0)),
        ],
        out_specs=pl.BlockSpec((B, D), lambda i: (0, 0)),
        out_shape=jax.ShapeDtypeStruct((B, D), jnp.float32),
    )(ids2, alpha2, h)


def kernel(h, batch, Wg, bg):
    h = h.astype(jnp.float32)
    ids = batch.astype(jnp.int32)
    wg_row = Wg.astype(jnp.float32).reshape(1, D)
    bg2 = bg.astype(jnp.float32).reshape(1, 1)

    gate, gmax = _gate_pass(h, wg_row, bg2)
    gmax_vec = jnp.broadcast_to(gmax.reshape(()), (16,))
    alpha = _alpha_pass(gate.reshape(N), ids, gmax_vec)
    out = _pool_pass(ids.reshape(N, 1), alpha.reshape(N, 1), h)
    return out


# speedup vs baseline: 6.5744x; 6.5744x over previous
import functools

import jax
import jax.numpy as jnp
from jax import lax
from jax.experimental import pallas as pl
from jax.experimental.pallas import tpu as pltpu
from jax.experimental.pallas import tpu_sc as plsc

N = 320000
D = 128
B = 1024

BA = 6400
BN = 2560
assert N % BA == 0 and N % BN == 0
NSC = 16
CHUNK = N // NSC
ITERS = CHUNK // 16


def _gate_kernel(wg_ref, bg_ref, h_ref, gate_ref, gmax_ref):
    g = jnp.sum(h_ref[...] * wg_ref[...], axis=1, keepdims=True) + bg_ref[0, 0]
    gate_ref[...] = g
    bm = jnp.max(g)

    @pl.when(pl.program_id(0) == 0)
    def _():
        gmax_ref[0, 0] = bm

    @pl.when(pl.program_id(0) > 0)
    def _():
        gmax_ref[0, 0] = jnp.maximum(gmax_ref[0, 0], bm)


def _gate_pass(h, wg_row, bg2):
    return pl.pallas_call(
        _gate_kernel,
        grid=(N // BA,),
        in_specs=[
            pl.BlockSpec((1, D), lambda i: (0, 0)),
            pl.BlockSpec(memory_space=pltpu.SMEM),
            pl.BlockSpec((BA, D), lambda i: (i, 0)),
        ],
        out_specs=[
            pl.BlockSpec((BA, 1), lambda i: (i, 0)),
            pl.BlockSpec(memory_space=pltpu.SMEM),
        ],
        out_shape=[
            jax.ShapeDtypeStruct((N, 1), jnp.float32),
            jax.ShapeDtypeStruct((1, 1), jnp.float32),
        ],
    )(wg_row, bg2, h)


def _alpha_kernel(gate_hbm, batch_hbm, gmax_hbm, alpha_hbm,
                  g_v, b_v, sloc_v, stot_v, sall_v, gm_v, sh_shared):
    sid = lax.axis_index("s")
    base = sid * CHUNK
    pltpu.sync_copy(gate_hbm.at[pl.ds(base, CHUNK)], g_v)
    pltpu.sync_copy(batch_hbm.at[pl.ds(base, CHUNK)], b_v)
    pltpu.sync_copy(gmax_hbm, gm_v)
    gmax = gm_v[...]

    zeros = jnp.zeros((16,), jnp.float32)

    def zbody(j, carry):
        sloc_v[pl.ds(j * 16, 16)] = zeros
        return carry

    lax.fori_loop(0, B // 16, zbody, 0)

    def body1(i, carry):
        off = i * 16
        e = jnp.exp(g_v[pl.ds(off, 16)] - gmax)
        ids = b_v[pl.ds(off, 16)]
        g_v[pl.ds(off, 16)] = e
        plsc.addupdate_scatter(sloc_v, [ids], e)
        return carry

    lax.fori_loop(0, ITERS, body1, 0)

    pltpu.sync_copy(sloc_v, sh_shared.at[sid])
    plsc.subcore_barrier()
    pltpu.sync_copy(sh_shared, sall_v)

    def cbody(j, carry):
        off = j * 16
        acc = sall_v[0, pl.ds(off, 16)]
        for w in range(1, NSC):
            acc = acc + sall_v[w, pl.ds(off, 16)]
        stot_v[pl.ds(off, 16)] = acc
        return carry

    lax.fori_loop(0, B // 16, cbody, 0)

    def body2(i, carry):
        off = i * 16
        e = g_v[pl.ds(off, 16)]
        ids = b_v[pl.ds(off, 16)]
        s = plsc.load_gather(stot_v, [ids])
        g_v[pl.ds(off, 16)] = e / (s + 1e-8)
        return carry

    lax.fori_loop(0, ITERS, body2, 0)

    pltpu.sync_copy(g_v, alpha_hbm.at[pl.ds(base, CHUNK)])


@functools.cache
def _build_alpha_pass():
    return pl.kernel(
        _alpha_kernel,
        mesh=plsc.VectorSubcoreMesh(core_axis_name="c", subcore_axis_name="s",
                                    num_cores=1),
        out_type=jax.ShapeDtypeStruct((N,), jnp.float32),
        compiler_params=pltpu.CompilerParams(needs_layout_passes=False),
        scratch_types=[
            pltpu.VMEM((CHUNK,), jnp.float32),
            pltpu.VMEM((CHUNK,), jnp.int32),
            pltpu.VMEM((B,), jnp.float32),
            pltpu.VMEM((B,), jnp.float32),
            pltpu.VMEM((NSC, B), jnp.float32),
            pltpu.VMEM((16,), jnp.float32),
            pltpu.VMEM_SHARED((NSC, B), jnp.float32),
        ],
    )


def _alpha_pass(gate_flat, ids, gmax_vec):
    return _build_alpha_pass()(gate_flat, ids, gmax_vec)


def _pool_kernel(ids_ref, alpha_ref, h_ref, out_ref):
    @pl.when(pl.program_id(0) == 0)
    def _():
        out_ref[...] = jnp.zeros_like(out_ref)

    w = h_ref[...] * alpha_ref[...]
    ids = ids_ref[...]
    lo = jnp.min(ids) // 128
    hi = jnp.max(ids) // 128
    iota = lax.broadcasted_iota(jnp.int32, (BN, 128), 1)

    def chunk_body(k, carry):
        seg0 = (lo + k) * 128
        m = (ids == (iota + seg0)).astype(jnp.float32)
        part = lax.dot_general(m, w, (((0,), (0,)), ((), ())),
                               preferred_element_type=jnp.float32)
        out_ref[pl.ds(seg0, 128), :] += part
        return carry

    lax.fori_loop(0, hi - lo + 1, chunk_body, 0)


def _pool_pass(ids2, alpha2, h):
    return pl.pallas_call(
        _pool_kernel,
        grid=(N // BN,),
        in_specs=[
            pl.BlockSpec((BN, 1), lambda i: (i, 0)),
            pl.BlockSpec((BN, 1), lambda i: (i, 0)),
            pl.BlockSpec((BN, D), lambda i: (i, 0)),
        ],
        out_specs=pl.BlockSpec((B, D), lambda i: (0, 0)),
        out_shape=jax.ShapeDtypeStruct((B, D), jnp.float32),
    )(ids2, alpha2, h)


def kernel(h, batch, Wg, bg):
    h = h.astype(jnp.float32)
    ids = batch.astype(jnp.int32)
    wg_row = Wg.astype(jnp.float32).reshape(1, D)
    bg2 = bg.astype(jnp.float32).reshape(1, 1)

    gate, gmax = _gate_pass(h, wg_row, bg2)
    gmax_vec = jnp.broadcast_to(gmax.reshape(()), (16,))
    alpha = _alpha_pass(gate.reshape(N), ids, gmax_vec)
    out = _pool_pass(ids.reshape(N, 1), alpha.reshape(N, 1), h)
    return out


# speedup vs baseline: 6.6430x; 1.0104x over previous
import functools

import jax
import jax.numpy as jnp
from jax import lax
from jax.experimental import pallas as pl
from jax.experimental.pallas import tpu as pltpu
from jax.experimental.pallas import tpu_sc as plsc

N = 320000
D = 128
B = 1024

BA = 6400
BN = 2560
assert N % BA == 0 and N % BN == 0
NSC = 16
CHUNK = N // NSC
ITERS = CHUNK // 16


def _gate_kernel(wg_ref, bg_ref, h_ref, gate_ref, gmax_ref):
    g = jnp.sum(h_ref[...] * wg_ref[...], axis=1, keepdims=True) + bg_ref[0, 0]
    gate_ref[...] = g
    bm = jnp.max(g)

    @pl.when(pl.program_id(0) == 0)
    def _():
        gmax_ref[0, 0] = bm

    @pl.when(pl.program_id(0) > 0)
    def _():
        gmax_ref[0, 0] = jnp.maximum(gmax_ref[0, 0], bm)


def _gate_pass(h, wg_row, bg2):
    return pl.pallas_call(
        _gate_kernel,
        grid=(N // BA,),
        in_specs=[
            pl.BlockSpec((1, D), lambda i: (0, 0)),
            pl.BlockSpec(memory_space=pltpu.SMEM),
            pl.BlockSpec((BA, D), lambda i: (i, 0)),
        ],
        out_specs=[
            pl.BlockSpec((BA, 1), lambda i: (i, 0)),
            pl.BlockSpec(memory_space=pltpu.SMEM),
        ],
        out_shape=[
            jax.ShapeDtypeStruct((N, 1), jnp.float32),
            jax.ShapeDtypeStruct((1, 1), jnp.float32),
        ],
    )(wg_row, bg2, h)


def _alpha_kernel(gate_hbm, batch_hbm, gmax_hbm, alpha_hbm,
                  g_v, b_v, sloc_v, stot_v, sall_v, gm_v, sh_shared):
    sid = lax.axis_index("s")
    base = sid * CHUNK
    pltpu.sync_copy(gate_hbm.at[pl.ds(base, CHUNK)], g_v)
    pltpu.sync_copy(batch_hbm.at[pl.ds(base, CHUNK)], b_v)
    pltpu.sync_copy(gmax_hbm, gm_v)
    gmax = gm_v[...]

    zeros = jnp.zeros((16,), jnp.float32)

    def zbody(j, carry):
        sloc_v[pl.ds(j * 16, 16)] = zeros
        return carry

    lax.fori_loop(0, B // 16, zbody, 0)

    def body1(i, carry):
        off = i * 16
        e = jnp.exp(g_v[pl.ds(off, 16)] - gmax)
        ids = b_v[pl.ds(off, 16)]
        g_v[pl.ds(off, 16)] = e
        plsc.addupdate_scatter(sloc_v, [ids], e)
        return carry

    lax.fori_loop(0, ITERS, body1, 0)

    pltpu.sync_copy(sloc_v, sh_shared.at[sid])
    plsc.subcore_barrier()
    pltpu.sync_copy(sh_shared, sall_v)

    def cbody(j, carry):
        off = j * 16
        acc = sall_v[0, pl.ds(off, 16)]
        for w in range(1, NSC):
            acc = acc + sall_v[w, pl.ds(off, 16)]
        stot_v[pl.ds(off, 16)] = acc
        return carry

    lax.fori_loop(0, B // 16, cbody, 0)

    def body2(i, carry):
        off = i * 16
        e = g_v[pl.ds(off, 16)]
        ids = b_v[pl.ds(off, 16)]
        s = plsc.load_gather(stot_v, [ids])
        g_v[pl.ds(off, 16)] = e / (s + 1e-8)
        return carry

    lax.fori_loop(0, ITERS, body2, 0)

    pltpu.sync_copy(g_v, alpha_hbm.at[pl.ds(base, CHUNK)])


@functools.cache
def _build_alpha_pass():
    return pl.kernel(
        _alpha_kernel,
        mesh=plsc.VectorSubcoreMesh(core_axis_name="c", subcore_axis_name="s",
                                    num_cores=1),
        out_type=jax.ShapeDtypeStruct((N,), jnp.float32),
        compiler_params=pltpu.CompilerParams(needs_layout_passes=False),
        scratch_types=[
            pltpu.VMEM((CHUNK,), jnp.float32),
            pltpu.VMEM((CHUNK,), jnp.int32),
            pltpu.VMEM((B,), jnp.float32),
            pltpu.VMEM((B,), jnp.float32),
            pltpu.VMEM((NSC, B), jnp.float32),
            pltpu.VMEM((16,), jnp.float32),
            pltpu.VMEM_SHARED((NSC, B), jnp.float32),
        ],
    )


def _alpha_pass(gate_flat, ids, gmax_vec):
    return _build_alpha_pass()(gate_flat, ids, gmax_vec)


def _pool_kernel(info_ref, ids_ref, alpha_ref, h_ref, out_ref):
    i = pl.program_id(0)

    @pl.when(i == 0)
    def _():
        out_ref[...] = jnp.zeros_like(out_ref)

    lo = info_ref[0, i]
    hi = info_ref[1, i]
    ids = ids_ref[...]
    alpha = alpha_ref[...]
    h = h_ref[...]
    iota = lax.broadcasted_iota(jnp.int32, (BN, 128), 1)

    def chunk_body(k, carry):
        seg0 = (lo + k) * 128
        m = jnp.where(ids == (iota + seg0), alpha, 0.0)
        part = lax.dot_general(m, h, (((0,), (0,)), ((), ())),
                               preferred_element_type=jnp.float32)
        out_ref[pl.ds(seg0, 128), :] += part
        return carry

    lax.fori_loop(0, hi - lo + 1, chunk_body, 0)


def _pool_pass(info, ids2, alpha2, h):
    return pl.pallas_call(
        _pool_kernel,
        grid=(N // BN,),
        in_specs=[
            pl.BlockSpec(memory_space=pltpu.SMEM),
            pl.BlockSpec((BN, 1), lambda i: (i, 0)),
            pl.BlockSpec((BN, 1), lambda i: (i, 0)),
            pl.BlockSpec((BN, D), lambda i: (i, 0)),
        ],
        out_specs=pl.BlockSpec((B, D), lambda i: (0, 0)),
        out_shape=jax.ShapeDtypeStruct((B, D), jnp.float32),
    )(info, ids2, alpha2, h)


def kernel(h, batch, Wg, bg):
    h = h.astype(jnp.float32)
    ids = batch.astype(jnp.int32)
    wg_row = Wg.astype(jnp.float32).reshape(1, D)
    bg2 = bg.astype(jnp.float32).reshape(1, 1)

    gate, gmax = _gate_pass(h, wg_row, bg2)
    gmax_vec = jnp.broadcast_to(gmax.reshape(()), (16,))
    alpha = _alpha_pass(gate.reshape(N), ids, gmax_vec)
    info = jnp.stack([ids[::BN] // 128, ids[BN - 1::BN] // 128])
    out = _pool_pass(info, ids.reshape(N, 1), alpha.reshape(N, 1), h)
    return out


# speedup vs baseline: 11.9614x; 1.8006x over previous
import functools

import jax
import jax.numpy as jnp
from jax import lax
from jax.experimental import pallas as pl
from jax.experimental.pallas import tpu as pltpu
from jax.experimental.pallas import tpu_sc as plsc

N = 320000
D = 128
B = 1024

BA = 6400
BN = 2560
assert N % BA == 0 and N % BN == 0
NSC = 16
CHUNK = N // NSC
ITERS = CHUNK // 16


def _gate_kernel(wg_ref, bg_ref, h_ref, gate_ref, gmax_ref):
    g = jnp.sum(h_ref[...] * wg_ref[...], axis=1, keepdims=True) + bg_ref[0, 0]
    gate_ref[...] = g
    bm = jnp.max(g)

    @pl.when(pl.program_id(0) == 0)
    def _():
        gmax_ref[0, 0] = bm

    @pl.when(pl.program_id(0) > 0)
    def _():
        gmax_ref[0, 0] = jnp.maximum(gmax_ref[0, 0], bm)


def _gate_pass(h, wg_row, bg2):
    return pl.pallas_call(
        _gate_kernel,
        grid=(N // BA,),
        in_specs=[
            pl.BlockSpec((1, D), lambda i: (0, 0)),
            pl.BlockSpec(memory_space=pltpu.SMEM),
            pl.BlockSpec((BA, D), lambda i: (i, 0)),
        ],
        out_specs=[
            pl.BlockSpec((BA, 1), lambda i: (i, 0)),
            pl.BlockSpec(memory_space=pltpu.SMEM),
        ],
        out_shape=[
            jax.ShapeDtypeStruct((N, 1), jnp.float32),
            jax.ShapeDtypeStruct((1, 1), jnp.float32),
        ],
    )(wg_row, bg2, h)


def _alpha_kernel(gate_hbm, batch_hbm, gmax_hbm, alpha_hbm,
                  g_v, b_v, sloc_v, stot_v, sall_v, gm_v, sh_shared):
    sid = lax.axis_index("s")
    base = sid * CHUNK
    pltpu.sync_copy(gate_hbm.at[pl.ds(base, CHUNK)], g_v)
    pltpu.sync_copy(batch_hbm.at[pl.ds(base, CHUNK)], b_v)
    pltpu.sync_copy(gmax_hbm, gm_v)
    gmax = gm_v[...]

    zeros = jnp.zeros((16,), jnp.float32)

    def zbody(j, carry):
        sloc_v[pl.ds(j * 16, 16)] = zeros
        return carry

    lax.fori_loop(0, B // 16, zbody, 0)

    def body1(i, carry):
        off = i * 16
        e = jnp.exp(g_v[pl.ds(off, 16)] - gmax)
        ids = b_v[pl.ds(off, 16)]
        g_v[pl.ds(off, 16)] = e
        plsc.addupdate_scatter(sloc_v, [ids], e)
        return carry

    lax.fori_loop(0, ITERS, body1, 0)

    pltpu.sync_copy(sloc_v, sh_shared.at[sid])
    plsc.subcore_barrier()
    pltpu.sync_copy(sh_shared, sall_v)

    def cbody(j, carry):
        off = j * 16
        acc = sall_v[0, pl.ds(off, 16)]
        for w in range(1, NSC):
            acc = acc + sall_v[w, pl.ds(off, 16)]
        stot_v[pl.ds(off, 16)] = acc
        return carry

    lax.fori_loop(0, B // 16, cbody, 0)

    def body2(i, carry):
        off = i * 16
        e = g_v[pl.ds(off, 16)]
        ids = b_v[pl.ds(off, 16)]
        s = plsc.load_gather(stot_v, [ids])
        g_v[pl.ds(off, 16)] = e / (s + 1e-8)
        return carry

    lax.fori_loop(0, ITERS, body2, 0)

    pltpu.sync_copy(g_v, alpha_hbm.at[pl.ds(base, CHUNK)])


@functools.cache
def _build_alpha_pass():
    return pl.kernel(
        _alpha_kernel,
        mesh=plsc.VectorSubcoreMesh(core_axis_name="c", subcore_axis_name="s",
                                    num_cores=1),
        out_type=jax.ShapeDtypeStruct((N,), jnp.float32),
        compiler_params=pltpu.CompilerParams(needs_layout_passes=False),
        scratch_types=[
            pltpu.VMEM((CHUNK,), jnp.float32),
            pltpu.VMEM((CHUNK,), jnp.int32),
            pltpu.VMEM((B,), jnp.float32),
            pltpu.VMEM((B,), jnp.float32),
            pltpu.VMEM((NSC, B), jnp.float32),
            pltpu.VMEM((16,), jnp.float32),
            pltpu.VMEM_SHARED((NSC, B), jnp.float32),
        ],
    )


def _alpha_pass(gate_flat, ids, gmax_vec):
    return _build_alpha_pass()(gate_flat, ids, gmax_vec)


def _pool_kernel(info_ref, ids_ref, alpha_ref, h_ref, out_ref):
    i = pl.program_id(0)

    @pl.when(i == 0)
    def _():
        out_ref[...] = jnp.zeros_like(out_ref)

    lo = info_ref[0, i]
    hi = info_ref[1, i]
    ids = ids_ref[0]
    alpha = alpha_ref[0]
    h = h_ref[...]
    seg_iota = lax.broadcasted_iota(jnp.int32, (128, BN), 0)

    def chunk_body(k, carry):
        seg0 = (lo + k) * 128
        m = jnp.where(ids == (seg_iota + seg0), alpha, 0.0)
        part = lax.dot_general(m, h, (((1,), (0,)), ((), ())),
                               preferred_element_type=jnp.float32)
        out_ref[pl.ds(seg0, 128), :] += part
        return carry

    lax.fori_loop(0, hi - lo + 1, chunk_body, 0)


def _pool_pass(info, ids3, alpha3, h):
    return pl.pallas_call(
        _pool_kernel,
        grid=(N // BN,),
        in_specs=[
            pl.BlockSpec(memory_space=pltpu.SMEM),
            pl.BlockSpec((1, 1, BN), lambda i: (i, 0, 0)),
            pl.BlockSpec((1, 1, BN), lambda i: (i, 0, 0)),
            pl.BlockSpec((BN, D), lambda i: (i, 0)),
        ],
        out_specs=pl.BlockSpec((B, D), lambda i: (0, 0)),
        out_shape=jax.ShapeDtypeStruct((B, D), jnp.float32),
    )(info, ids3, alpha3, h)


def kernel(h, batch, Wg, bg):
    h = h.astype(jnp.float32)
    ids = batch.astype(jnp.int32)
    wg_row = Wg.astype(jnp.float32).reshape(1, D)
    bg2 = bg.astype(jnp.float32).reshape(1, 1)

    gate, gmax = _gate_pass(h, wg_row, bg2)
    gmax_vec = jnp.broadcast_to(gmax.reshape(()), (16,))
    alpha = _alpha_pass(gate.reshape(N), ids, gmax_vec)
    info = jnp.stack([ids[::BN] // 128, ids[BN - 1::BN] // 128])
    out = _pool_pass(info, ids.reshape(N // BN, 1, BN),
                     alpha.reshape(N // BN, 1, BN), h)
    return out


# speedup vs baseline: 15.7753x; 1.3189x over previous
import functools

import jax
import jax.numpy as jnp
from jax import lax
from jax.experimental import pallas as pl
from jax.experimental.pallas import tpu as pltpu
from jax.experimental.pallas import tpu_sc as plsc

N = 320000
D = 128
B = 1024

BA = 6400
BN = 2560
assert N % BA == 0 and N % BN == 0
NSC = 16
CHUNK = N // NSC
ITERS = CHUNK // 16


def _gate_kernel(wg_ref, bg_ref, h_ref, gate_ref, gmax_ref):
    g = lax.dot_general(wg_ref[...], h_ref[...], (((1,), (1,)), ((), ())),
                        preferred_element_type=jnp.float32) + bg_ref[0, 0]
    gate_ref[0] = g
    bm = jnp.max(g)

    @pl.when(pl.program_id(0) == 0)
    def _():
        gmax_ref[0, 0] = bm

    @pl.when(pl.program_id(0) > 0)
    def _():
        gmax_ref[0, 0] = jnp.maximum(gmax_ref[0, 0], bm)


def _gate_pass(h, wg_row, bg2):
    return pl.pallas_call(
        _gate_kernel,
        grid=(N // BA,),
        in_specs=[
            pl.BlockSpec((1, D), lambda i: (0, 0)),
            pl.BlockSpec(memory_space=pltpu.SMEM),
            pl.BlockSpec((BA, D), lambda i: (i, 0)),
        ],
        out_specs=[
            pl.BlockSpec((1, 1, BA), lambda i: (i, 0, 0)),
            pl.BlockSpec(memory_space=pltpu.SMEM),
        ],
        out_shape=[
            jax.ShapeDtypeStruct((N // BA, 1, BA), jnp.float32),
            jax.ShapeDtypeStruct((1, 1), jnp.float32),
        ],
    )(wg_row, bg2, h)


def _alpha_kernel(gate_hbm, batch_hbm, gmax_hbm, alpha_hbm,
                  g_v, b_v, sloc_v, stot_v, sall_v, gm_v, sh_shared):
    sid = lax.axis_index("s")
    base = sid * CHUNK
    pltpu.sync_copy(gate_hbm.at[pl.ds(base, CHUNK)], g_v)
    pltpu.sync_copy(batch_hbm.at[pl.ds(base, CHUNK)], b_v)
    pltpu.sync_copy(gmax_hbm, gm_v)
    gmax = gm_v[...]

    zeros = jnp.zeros((16,), jnp.float32)

    def zbody(j, carry):
        sloc_v[pl.ds(j * 16, 16)] = zeros
        return carry

    lax.fori_loop(0, B // 16, zbody, 0)

    def body1(i, carry):
        off = i * 16
        e = jnp.exp(g_v[pl.ds(off, 16)] - gmax)
        ids = b_v[pl.ds(off, 16)]
        g_v[pl.ds(off, 16)] = e
        plsc.addupdate_scatter(sloc_v, [ids], e)
        return carry

    lax.fori_loop(0, ITERS, body1, 0)

    pltpu.sync_copy(sloc_v, sh_shared.at[sid])
    plsc.subcore_barrier()
    pltpu.sync_copy(sh_shared, sall_v)

    def cbody(j, carry):
        off = j * 16
        acc = sall_v[0, pl.ds(off, 16)]
        for w in range(1, NSC):
            acc = acc + sall_v[w, pl.ds(off, 16)]
        stot_v[pl.ds(off, 16)] = acc
        return carry

    lax.fori_loop(0, B // 16, cbody, 0)

    def body2(i, carry):
        off = i * 16
        e = g_v[pl.ds(off, 16)]
        ids = b_v[pl.ds(off, 16)]
        s = plsc.load_gather(stot_v, [ids])
        g_v[pl.ds(off, 16)] = e / (s + 1e-8)
        return carry

    lax.fori_loop(0, ITERS, body2, 0)

    pltpu.sync_copy(g_v, alpha_hbm.at[pl.ds(base, CHUNK)])


@functools.cache
def _build_alpha_pass():
    return pl.kernel(
        _alpha_kernel,
        mesh=plsc.VectorSubcoreMesh(core_axis_name="c", subcore_axis_name="s",
                                    num_cores=1),
        out_type=jax.ShapeDtypeStruct((N,), jnp.float32),
        compiler_params=pltpu.CompilerParams(needs_layout_passes=False),
        scratch_types=[
            pltpu.VMEM((CHUNK,), jnp.float32),
            pltpu.VMEM((CHUNK,), jnp.int32),
            pltpu.VMEM((B,), jnp.float32),
            pltpu.VMEM((B,), jnp.float32),
            pltpu.VMEM((NSC, B), jnp.float32),
            pltpu.VMEM((16,), jnp.float32),
            pltpu.VMEM_SHARED((NSC, B), jnp.float32),
        ],
    )


def _alpha_pass(gate_flat, ids, gmax_vec):
    return _build_alpha_pass()(gate_flat, ids, gmax_vec)


def _pool_kernel(info_ref, ids_ref, alpha_ref, h_ref, out_ref):
    i = pl.program_id(0)

    @pl.when(i == 0)
    def _():
        out_ref[...] = jnp.zeros_like(out_ref)

    lo = info_ref[0, i]
    hi = info_ref[1, i]
    ids = ids_ref[0]
    alpha = alpha_ref[0]
    h = h_ref[...]
    seg_iota = lax.broadcasted_iota(jnp.int32, (128, BN), 0)

    def chunk_body(k, carry):
        seg0 = (lo + k) * 128
        m = jnp.where(ids == (seg_iota + seg0), alpha, 0.0)
        part = lax.dot_general(m, h, (((1,), (0,)), ((), ())),
                               preferred_element_type=jnp.float32)
        out_ref[pl.ds(seg0, 128), :] += part
        return carry

    lax.fori_loop(0, hi - lo + 1, chunk_body, 0)


def _pool_pass(info, ids3, alpha3, h):
    return pl.pallas_call(
        _pool_kernel,
        grid=(N // BN,),
        in_specs=[
            pl.BlockSpec(memory_space=pltpu.SMEM),
            pl.BlockSpec((1, 1, BN), lambda i: (i, 0, 0)),
            pl.BlockSpec((1, 1, BN), lambda i: (i, 0, 0)),
            pl.BlockSpec((BN, D), lambda i: (i, 0)),
        ],
        out_specs=pl.BlockSpec((B, D), lambda i: (0, 0)),
        out_shape=jax.ShapeDtypeStruct((B, D), jnp.float32),
    )(info, ids3, alpha3, h)


def kernel(h, batch, Wg, bg):
    h = h.astype(jnp.float32)
    ids = batch.astype(jnp.int32)
    wg_row = Wg.astype(jnp.float32).reshape(1, D)
    bg2 = bg.astype(jnp.float32).reshape(1, 1)

    gate, gmax = _gate_pass(h, wg_row, bg2)
    gmax_vec = jnp.broadcast_to(gmax.reshape(()), (16,))
    alpha = _alpha_pass(gate.reshape(N), ids, gmax_vec)
    info = jnp.stack([ids[::BN] // 128, ids[BN - 1::BN] // 128])
    out = _pool_pass(info, ids.reshape(N // BN, 1, BN),
                     alpha.reshape(N // BN, 1, BN), h)
    return out


# speedup vs baseline: 19.8029x; 1.2553x over previous
import functools

import jax
import jax.numpy as jnp
from jax import lax
from jax.experimental import pallas as pl
from jax.experimental.pallas import tpu as pltpu
from jax.experimental.pallas import tpu_sc as plsc

N = 320000
D = 128
B = 1024

BA = 6400
BN = 2560
assert N % BA == 0 and N % BN == 0
NSC = 16
CHUNK = N // NSC
ITERS = CHUNK // 16


def _gate_kernel(wg_ref, bg_ref, h_ref, gate_ref, gmax_ref):
    g = lax.dot_general(wg_ref[...], h_ref[...], (((1,), (1,)), ((), ())),
                        preferred_element_type=jnp.float32) + bg_ref[0, 0]
    gate_ref[0] = g
    bm = jnp.max(g)

    @pl.when(pl.program_id(0) == 0)
    def _():
        gmax_ref[0, 0] = bm

    @pl.when(pl.program_id(0) > 0)
    def _():
        gmax_ref[0, 0] = jnp.maximum(gmax_ref[0, 0], bm)


def _gate_pass(h, wg_row, bg2):
    return pl.pallas_call(
        _gate_kernel,
        grid=(N // BA,),
        in_specs=[
            pl.BlockSpec((1, D), lambda i: (0, 0)),
            pl.BlockSpec(memory_space=pltpu.SMEM),
            pl.BlockSpec((BA, D), lambda i: (i, 0)),
        ],
        out_specs=[
            pl.BlockSpec((1, 1, BA), lambda i: (i, 0, 0)),
            pl.BlockSpec(memory_space=pltpu.SMEM),
        ],
        out_shape=[
            jax.ShapeDtypeStruct((N // BA, 1, BA), jnp.float32),
            jax.ShapeDtypeStruct((1, 1), jnp.float32),
        ],
    )(wg_row, bg2, h)


BROWS = B // NSC


def _segsum_kernel(gate_hbm, batch_hbm, gmax_hbm, sbc_hbm,
                   g_v, b_v, sloc_v, stot_v, sall_v, gm_v, sbc_v, sh_shared):
    sid = lax.axis_index("s")
    base = sid * CHUNK
    pltpu.sync_copy(gate_hbm.at[pl.ds(base, CHUNK)], g_v)
    pltpu.sync_copy(batch_hbm.at[pl.ds(base, CHUNK)], b_v)
    pltpu.sync_copy(gmax_hbm, gm_v)
    gmax = gm_v[...]

    zeros = jnp.zeros((16,), jnp.float32)

    def zbody(j, carry):
        sloc_v[pl.ds(j * 16, 16)] = zeros
        return carry

    lax.fori_loop(0, B // 16, zbody, 0)

    def body1(i, carry):
        off = i * 16
        e = jnp.exp(g_v[pl.ds(off, 16)] - gmax)
        ids = b_v[pl.ds(off, 16)]
        plsc.addupdate_scatter(sloc_v, [ids], e)
        return carry

    lax.fori_loop(0, ITERS, body1, 0)

    pltpu.sync_copy(sloc_v, sh_shared.at[sid])
    plsc.subcore_barrier()
    pltpu.sync_copy(sh_shared, sall_v)

    def cbody(j, carry):
        off = j * 16
        acc = sall_v[0, pl.ds(off, 16)]
        for w in range(1, NSC):
            acc = acc + sall_v[w, pl.ds(off, 16)]
        stot_v[pl.ds(off, 16)] = acc
        return carry

    lax.fori_loop(0, B // 16, cbody, 0)

    def bbody(r, carry):
        seg = sid * BROWS + r
        vec = plsc.load_gather(stot_v, [jnp.full((16,), seg, jnp.int32)])
        for j in range(128 // 16):
            sbc_v[r, pl.ds(j * 16, 16)] = vec
        return carry

    lax.fori_loop(0, BROWS, bbody, 0)

    pltpu.sync_copy(sbc_v, sbc_hbm.at[pl.ds(sid * BROWS, BROWS)])


@functools.cache
def _build_segsum_pass():
    return pl.kernel(
        _segsum_kernel,
        mesh=plsc.VectorSubcoreMesh(core_axis_name="c", subcore_axis_name="s",
                                    num_cores=1),
        out_type=jax.ShapeDtypeStruct((B, D), jnp.float32),
        compiler_params=pltpu.CompilerParams(needs_layout_passes=False),
        scratch_types=[
            pltpu.VMEM((CHUNK,), jnp.float32),
            pltpu.VMEM((CHUNK,), jnp.int32),
            pltpu.VMEM((B,), jnp.float32),
            pltpu.VMEM((B,), jnp.float32),
            pltpu.VMEM((NSC, B), jnp.float32),
            pltpu.VMEM((16,), jnp.float32),
            pltpu.VMEM((BROWS, D), jnp.float32),
            pltpu.VMEM_SHARED((NSC, B), jnp.float32),
        ],
    )


def _segsum_pass(gate_flat, ids, gmax_vec):
    return _build_segsum_pass()(gate_flat, ids, gmax_vec)


def _pool_kernel(info_ref, gmax_ref, ids_ref, gate_ref, h_ref, out_ref):
    i = pl.program_id(0)

    @pl.when(i == 0)
    def _():
        out_ref[...] = jnp.zeros_like(out_ref)

    lo = info_ref[0, i]
    hi = info_ref[1, i]
    ids = ids_ref[0]
    e = jnp.exp(gate_ref[0] - gmax_ref[0, 0])
    h = h_ref[...]
    seg_iota = lax.broadcasted_iota(jnp.int32, (128, BN), 0)

    def chunk_body(k, carry):
        seg0 = (lo + k) * 128
        m = jnp.where(ids == (seg_iota + seg0), e, 0.0)
        part = lax.dot_general(m, h, (((1,), (0,)), ((), ())),
                               preferred_element_type=jnp.float32)
        out_ref[pl.ds(seg0, 128), :] += part
        return carry

    lax.fori_loop(0, hi - lo + 1, chunk_body, 0)


def _pool_pass(info, gmax, ids3, gate3, h):
    return pl.pallas_call(
        _pool_kernel,
        grid=(N // BN,),
        in_specs=[
            pl.BlockSpec(memory_space=pltpu.SMEM),
            pl.BlockSpec(memory_space=pltpu.SMEM),
            pl.BlockSpec((1, 1, BN), lambda i: (i, 0, 0)),
            pl.BlockSpec((1, 1, BN), lambda i: (i, 0, 0)),
            pl.BlockSpec((BN, D), lambda i: (i, 0)),
        ],
        out_specs=pl.BlockSpec((B, D), lambda i: (0, 0)),
        out_shape=jax.ShapeDtypeStruct((B, D), jnp.float32),
    )(info, gmax, ids3, gate3, h)


def _div_kernel(acc_ref, s_ref, out_ref):
    out_ref[...] = acc_ref[...] / (s_ref[...] + 1e-8)


def _div_pass(acc, sbc):
    return pl.pallas_call(
        _div_kernel,
        out_shape=jax.ShapeDtypeStruct((B, D), jnp.float32),
    )(acc, sbc)


def kernel(h, batch, Wg, bg):
    h = h.astype(jnp.float32)
    ids = batch.astype(jnp.int32)
    wg_row = Wg.astype(jnp.float32).reshape(1, D)
    bg2 = bg.astype(jnp.float32).reshape(1, 1)

    gate, gmax = _gate_pass(h, wg_row, bg2)
    gmax_vec = jnp.broadcast_to(gmax.reshape(()), (16,))
    sbc = _segsum_pass(gate.reshape(N), ids, gmax_vec)
    info = jnp.stack([ids[::BN] // 128, ids[BN - 1::BN] // 128])
    acc = _pool_pass(info, gmax, ids.reshape(N // BN, 1, BN),
                     gate.reshape(N // BN, 1, BN), h)
    return _div_pass(acc, sbc)
